# baseline (speedup 1.0000x reference)
import functools

import jax
import jax.numpy as jnp
from jax import lax
from jax.experimental import pallas as pl
from jax.experimental.pallas import tpu as pltpu
from jax.experimental.pallas import tpu_sc as plsc

B = 16384
CONT_F = 13
CATE_F = 26
COMB_F = 325
COMB_V = 1000000
GATH_F = COMB_F + CATE_F
GATH_P = 352
CONT_P = 16
H1 = 176

NC = 2
NS = 16
NW = NC * NS
ROWS_PER_W = B // NW
CHUNK = 128
NCHUNK = ROWS_PER_W // CHUNK
NG = CHUNK // 16

_mesh = plsc.VectorSubcoreMesh(core_axis_name="c", subcore_axis_name="s")


@functools.partial(
    pl.kernel,
    mesh=_mesh,
    out_type=jax.ShapeDtypeStruct((B,), jnp.float32),
    scratch_types=[
        pltpu.VMEM((GATH_P, CHUNK), jnp.int32),
        pltpu.VMEM((GATH_P, CHUNK), jnp.float32),
        pltpu.VMEM((2 * CONT_P, CHUNK), jnp.float32),
        pltpu.VMEM((CONT_F * 16,), jnp.float32),
        pltpu.VMEM((CHUNK,), jnp.float32),
        pltpu.VMEM((ROWS_PER_W,), jnp.float32),
        pltpu.SemaphoreType.DMA,
        pltpu.SemaphoreType.DMA,
        pltpu.SemaphoreType.DMA,
    ],
)
def _poly2_sc(m_t, m_tf, tab, wbc_hbm, out_hbm,
              idx_v, val_v, cont_v, w_v, part_v, out_v,
              sem_a, sem_b, sem_s):
    cid = lax.axis_index("c")
    sid = lax.axis_index("s")
    wid = sid * NC + cid
    base = wid * ROWS_PER_W

    pltpu.sync_copy(wbc_hbm, w_v)

    def stage(c, parity, sync):
        rb = base + c * CHUNK
        copies = [
            (m_t.at[pl.ds(0, GATH_P), pl.ds(rb, CHUNK)], idx_v),
            (m_tf.at[pl.ds(GATH_P, CONT_P), pl.ds(rb, CHUNK)],
             cont_v.at[pl.ds(parity * CONT_P, CONT_P), :]),
        ]
        if sync:
            for src, dst in copies:
                pltpu.sync_copy(src, dst)
        else:
            for src, dst in copies:
                pltpu.async_copy(src, dst, sem_s)

    def stage_drain():
        pltpu.make_async_copy(m_t.at[pl.ds(0, GATH_P), pl.ds(0, CHUNK)],
                              idx_v, sem_s).wait()
        pltpu.make_async_copy(m_tf.at[pl.ds(GATH_P, CONT_P), pl.ds(0, CHUNK)],
                              cont_v.at[pl.ds(0, CONT_P), :], sem_s).wait()

    def fire(lo, hi, sem):
        def body(f, carry):
            pltpu.async_copy(tab.at[idx_v.at[f]], val_v.at[f], sem)
            return carry
        lax.fori_loop(lo, hi, body, jnp.int32(0))

    def drain(n_rows, sem):
        def body(f, carry):
            pltpu.make_async_copy(tab.at[pl.ds(0, CHUNK)],
                                  val_v.at[f], sem).wait()
            return carry
        lax.fori_loop(0, n_rows, body, jnp.int32(0))

    def compute_h1():
        def group(bs, carry):
            so = bs * 16

            def body(f, acc):
                return acc + val_v[f, pl.ds(so, 16)]

            acc = lax.fori_loop(0, H1, body, jnp.zeros((16,), jnp.float32))
            part_v[pl.ds(so, 16)] = acc
            return carry
        lax.fori_loop(0, NG, group, jnp.int32(0))

    def compute_h2(c, parity):
        def group(bs, carry):
            so = bs * 16

            def body(f, acc):
                return acc + val_v[f, pl.ds(so, 16)]

            acc = lax.fori_loop(H1, GATH_F, body, part_v[pl.ds(so, 16)])

            def body_w(f, acc):
                return acc + (cont_v[parity * CONT_P + f, pl.ds(so, 16)]
                              * w_v[pl.ds(f * 16, 16)])

            acc = lax.fori_loop(0, CONT_F, body_w, acc)

            out_v[pl.ds(c * CHUNK + so, 16)] = 1.0 / (1.0 + jnp.exp(-acc))
            return carry
        lax.fori_loop(0, NG, group, jnp.int32(0))

    stage(0, 0, sync=True)
    fire(0, H1, sem_a)
    for c in range(NCHUNK):
        parity = c % 2
        fire(H1, GATH_P, sem_b)
        drain(H1, sem_a)
        compute_h1()
        drain(GATH_P - H1, sem_b)
        if c < NCHUNK - 1:
            stage(c + 1, 1 - parity, sync=False)
            stage_drain()
            fire(0, H1, sem_a)
        compute_h2(c, parity)

    pltpu.sync_copy(out_v, out_hbm.at[pl.ds(base, ROWS_PER_W)])


def kernel(conts, cates, combs, cate_table, comb_table):
    wbc = jnp.repeat(cate_table[:CONT_F, 0], 16)
    m = jnp.concatenate(
        [combs.astype(jnp.int32),
         cates.astype(jnp.int32) + COMB_V,
         jnp.zeros((B, GATH_P - GATH_F), jnp.int32),
         lax.bitcast_convert_type(conts, jnp.int32),
         jnp.zeros((B, CONT_P - CONT_F), jnp.int32)], axis=1)
    tab = jnp.concatenate([comb_table.reshape(-1), cate_table.reshape(-1)])
    m_t = m.T
    out = _poly2_sc(m_t, lax.bitcast_convert_type(m_t, jnp.float32),
                    tab, wbc)
    return out.reshape(B, 1)


# speedup vs baseline: 1.3631x; 1.3631x over previous
import functools

import jax
import jax.numpy as jnp
from jax import lax
from jax.experimental import pallas as pl
from jax.experimental.pallas import tpu as pltpu
from jax.experimental.pallas import tpu_sc as plsc

B = 16384
CONT_F = 13
CATE_F = 26
COMB_F = 325
CONT_P = 16
CATE_P = 32
COMB_P = 328
H1 = 176

NC = 2
NS = 16
NW = NC * NS
ROWS_PER_W = B // NW
CHUNK = 128
NCHUNK = ROWS_PER_W // CHUNK
NG = CHUNK // 16

_mesh = plsc.VectorSubcoreMesh(core_axis_name="c", subcore_axis_name="s")


@functools.partial(
    pl.kernel,
    mesh=_mesh,
    out_type=jax.ShapeDtypeStruct((B,), jnp.float32),
    scratch_types=[
        pltpu.VMEM((COMB_P, CHUNK), jnp.int32),
        pltpu.VMEM((COMB_F, CHUNK), jnp.float32),
        pltpu.VMEM((CATE_P, CHUNK), jnp.int32),
        pltpu.VMEM((CATE_F, CHUNK), jnp.float32),
        pltpu.VMEM((2 * CONT_P, CHUNK), jnp.float32),
        pltpu.VMEM((CONT_F * 16,), jnp.float32),
        pltpu.VMEM((CHUNK,), jnp.float32),
        pltpu.VMEM((ROWS_PER_W,), jnp.float32),
        pltpu.SemaphoreType.DMA,
        pltpu.SemaphoreType.DMA,
        pltpu.SemaphoreType.DMA,
    ],
)
def _poly2_sc(conts_t, cates_t, combs_t, cate_tab, comb_tab, wbc_hbm,
              out_hbm,
              comb_idx_v, comb_val_v, cate_idx_v, cate_val_v, cont_v,
              w_v, part_v, out_v, sem_a, sem_b, sem_s):
    cid = lax.axis_index("c")
    sid = lax.axis_index("s")
    wid = sid * NC + cid
    base = wid * ROWS_PER_W

    pltpu.sync_copy(wbc_hbm, w_v)

    def stage(c, parity, sync):
        rb = base + c * CHUNK
        copies = [
            (combs_t.at[:, pl.ds(rb, CHUNK)], comb_idx_v),
            (cates_t.at[:, pl.ds(rb, CHUNK)], cate_idx_v),
            (conts_t.at[:, pl.ds(rb, CHUNK)],
             cont_v.at[pl.ds(parity * CONT_P, CONT_P), :]),
        ]
        if sync:
            for src, dst in copies:
                pltpu.sync_copy(src, dst)
        else:
            for src, dst in copies:
                pltpu.async_copy(src, dst, sem_s)

    def stage_drain():
        pltpu.make_async_copy(combs_t.at[:, pl.ds(0, CHUNK)],
                              comb_idx_v, sem_s).wait()
        pltpu.make_async_copy(cates_t.at[:, pl.ds(0, CHUNK)],
                              cate_idx_v, sem_s).wait()
        pltpu.make_async_copy(conts_t.at[:, pl.ds(0, CHUNK)],
                              cont_v.at[pl.ds(0, CONT_P), :], sem_s).wait()

    def fire_h1(sem):
        def body(f, carry):
            pltpu.async_copy(comb_tab.at[comb_idx_v.at[f]],
                             comb_val_v.at[f], sem)
            return carry
        lax.fori_loop(0, H1, body, jnp.int32(0))

    def fire_h2(sem):
        def body(f, carry):
            pltpu.async_copy(comb_tab.at[comb_idx_v.at[f]],
                             comb_val_v.at[f], sem)
            return carry
        lax.fori_loop(H1, COMB_F, body, jnp.int32(0))

        def body_c(f, carry):
            pltpu.async_copy(cate_tab.at[cate_idx_v.at[f]],
                             cate_val_v.at[f], sem)
            return carry
        lax.fori_loop(0, CATE_F, body_c, jnp.int32(0))

    def drain(n_rows, sem):
        def body(f, carry):
            pltpu.make_async_copy(cate_tab.at[pl.ds(0, CHUNK)],
                                  comb_val_v.at[f], sem).wait()
            return carry
        lax.fori_loop(0, n_rows, body, jnp.int32(0))

    def compute_h1():
        def group(bs, carry):
            so = bs * 16

            def body(f, acc):
                return acc + comb_val_v[f, pl.ds(so, 16)]

            acc = lax.fori_loop(0, H1, body, jnp.zeros((16,), jnp.float32))
            part_v[pl.ds(so, 16)] = acc
            return carry
        lax.fori_loop(0, NG, group, jnp.int32(0))

    def compute_h2(c, parity):
        def group(bs, carry):
            so = bs * 16

            def body(f, acc):
                return acc + comb_val_v[f, pl.ds(so, 16)]

            acc = lax.fori_loop(H1, COMB_F, body, part_v[pl.ds(so, 16)])

            def body_c(f, acc):
                return acc + cate_val_v[f, pl.ds(so, 16)]

            acc = lax.fori_loop(0, CATE_F, body_c, acc)

            def body_w(f, acc):
                return acc + (cont_v[parity * CONT_P + f, pl.ds(so, 16)]
                              * w_v[pl.ds(f * 16, 16)])

            acc = lax.fori_loop(0, CONT_F, body_w, acc)

            out_v[pl.ds(c * CHUNK + so, 16)] = 1.0 / (1.0 + jnp.exp(-acc))
            return carry
        lax.fori_loop(0, NG, group, jnp.int32(0))

    stage(0, 0, sync=True)
    fire_h1(sem_a)
    for c in range(NCHUNK):
        parity = c % 2
        fire_h2(sem_b)
        drain(H1, sem_a)
        compute_h1()
        drain(COMB_F - H1 + CATE_F, sem_b)
        if c < NCHUNK - 1:
            stage(c + 1, 1 - parity, sync=False)
            stage_drain()
            fire_h1(sem_a)
        compute_h2(c, parity)

    pltpu.sync_copy(out_v, out_hbm.at[pl.ds(base, ROWS_PER_W)])


def _pad_t(x, fp, dtype):
    f = x.shape[1]
    return jnp.pad(x.astype(dtype), ((0, 0), (0, fp - f))).T


def kernel(conts, cates, combs, cate_table, comb_table):
    wbc = jnp.repeat(cate_table[:CONT_F, 0], 16)
    out = _poly2_sc(_pad_t(conts, CONT_P, jnp.float32),
                    _pad_t(cates, CATE_P, jnp.int32),
                    _pad_t(combs, COMB_P, jnp.int32),
                    cate_table.reshape(-1), comb_table.reshape(-1), wbc)
    return out.reshape(B, 1)


# speedup vs baseline: 1.4457x; 1.0606x over previous
import functools

import jax
import jax.numpy as jnp
from jax import lax
from jax.experimental import pallas as pl
from jax.experimental.pallas import tpu as pltpu
from jax.experimental.pallas import tpu_sc as plsc

B = 16384
CONT_F = 13
CATE_F = 26
COMB_F = 325
CONT_P = 13
CATE_P = 26
COMB_P = 325
H1 = 176

NC = 2
NS = 16
NW = NC * NS
ROWS_PER_W = B // NW
CHUNK = 128
NCHUNK = ROWS_PER_W // CHUNK
NG = CHUNK // 16

_mesh = plsc.VectorSubcoreMesh(core_axis_name="c", subcore_axis_name="s")


@functools.partial(
    pl.kernel,
    mesh=_mesh,
    out_type=jax.ShapeDtypeStruct((B,), jnp.float32),
    scratch_types=[
        pltpu.VMEM((COMB_P, CHUNK), jnp.int32),
        pltpu.VMEM((COMB_F, CHUNK), jnp.float32),
        pltpu.VMEM((CATE_P, CHUNK), jnp.int32),
        pltpu.VMEM((CATE_F, CHUNK), jnp.float32),
        pltpu.VMEM((2 * CONT_P, CHUNK), jnp.float32),
        pltpu.VMEM((CONT_F * 16,), jnp.float32),
        pltpu.VMEM((CHUNK,), jnp.float32),
        pltpu.VMEM((ROWS_PER_W,), jnp.float32),
        pltpu.SemaphoreType.DMA,
        pltpu.SemaphoreType.DMA,
        pltpu.SemaphoreType.DMA,
    ],
)
def _poly2_sc(conts_t, cates_t, combs_t, cate_tab, comb_tab, wbc_hbm,
              out_hbm,
              comb_idx_v, comb_val_v, cate_idx_v, cate_val_v, cont_v,
              w_v, part_v, out_v, sem_a, sem_b, sem_s):
    cid = lax.axis_index("c")
    sid = lax.axis_index("s")
    wid = sid * NC + cid
    base = wid * ROWS_PER_W

    pltpu.sync_copy(wbc_hbm, w_v)

    def stage(c, parity, sync):
        rb = base + c * CHUNK
        copies = [
            (combs_t.at[:, pl.ds(rb, CHUNK)], comb_idx_v),
            (cates_t.at[:, pl.ds(rb, CHUNK)], cate_idx_v),
            (conts_t.at[:, pl.ds(rb, CHUNK)],
             cont_v.at[pl.ds(parity * CONT_P, CONT_P), :]),
        ]
        if sync:
            for src, dst in copies:
                pltpu.sync_copy(src, dst)
        else:
            for src, dst in copies:
                pltpu.async_copy(src, dst, sem_s)

    def stage_drain():
        pltpu.make_async_copy(combs_t.at[:, pl.ds(0, CHUNK)],
                              comb_idx_v, sem_s).wait()
        pltpu.make_async_copy(cates_t.at[:, pl.ds(0, CHUNK)],
                              cate_idx_v, sem_s).wait()
        pltpu.make_async_copy(conts_t.at[:, pl.ds(0, CHUNK)],
                              cont_v.at[pl.ds(0, CONT_P), :], sem_s).wait()

    def fire_h1(sem):
        def body(f, carry):
            pltpu.async_copy(comb_tab.at[comb_idx_v.at[f]],
                             comb_val_v.at[f], sem)
            return carry
        lax.fori_loop(0, H1, body, jnp.int32(0))

    def fire_h2(sem):
        def body(f, carry):
            pltpu.async_copy(comb_tab.at[comb_idx_v.at[f]],
                             comb_val_v.at[f], sem)
            return carry
        lax.fori_loop(H1, COMB_F, body, jnp.int32(0))

        def body_c(f, carry):
            pltpu.async_copy(cate_tab.at[cate_idx_v.at[f]],
                             cate_val_v.at[f], sem)
            return carry
        lax.fori_loop(0, CATE_F, body_c, jnp.int32(0))

    def drain(n_rows, sem):
        def body(f, carry):
            pltpu.make_async_copy(cate_tab.at[pl.ds(0, CHUNK)],
                                  comb_val_v.at[f], sem).wait()
            return carry
        lax.fori_loop(0, n_rows, body, jnp.int32(0))

    def compute_h1():
        def group(bs, carry):
            so = bs * 16

            def body(f, acc):
                return acc + comb_val_v[f, pl.ds(so, 16)]

            acc = lax.fori_loop(0, H1, body, jnp.zeros((16,), jnp.float32))
            part_v[pl.ds(so, 16)] = acc
            return carry
        lax.fori_loop(0, NG, group, jnp.int32(0))

    def compute_h2(c, parity):
        def group(bs, carry):
            so = bs * 16

            def body(f, acc):
                return acc + comb_val_v[f, pl.ds(so, 16)]

            acc = lax.fori_loop(H1, COMB_F, body, part_v[pl.ds(so, 16)])

            def body_c(f, acc):
                return acc + cate_val_v[f, pl.ds(so, 16)]

            acc = lax.fori_loop(0, CATE_F, body_c, acc)

            def body_w(f, acc):
                return acc + (cont_v[parity * CONT_P + f, pl.ds(so, 16)]
                              * w_v[pl.ds(f * 16, 16)])

            acc = lax.fori_loop(0, CONT_F, body_w, acc)

            out_v[pl.ds(c * CHUNK + so, 16)] = 1.0 / (1.0 + jnp.exp(-acc))
            return carry
        lax.fori_loop(0, NG, group, jnp.int32(0))

    stage(0, 0, sync=True)
    fire_h1(sem_a)
    for c in range(NCHUNK):
        parity = c % 2
        fire_h2(sem_b)
        drain(H1, sem_a)
        compute_h1()
        drain(COMB_F - H1 + CATE_F, sem_b)
        if c < NCHUNK - 1:
            stage(c + 1, 1 - parity, sync=False)
            stage_drain()
            fire_h1(sem_a)
        compute_h2(c, parity)

    pltpu.sync_copy(out_v, out_hbm.at[pl.ds(base, ROWS_PER_W)])


def _pad_t(x, fp, dtype):
    f = x.shape[1]
    return jnp.pad(x.astype(dtype), ((0, 0), (0, fp - f))).T


def kernel(conts, cates, combs, cate_table, comb_table):
    wbc = jnp.repeat(cate_table[:CONT_F, 0], 16)
    out = _poly2_sc(_pad_t(conts, CONT_P, jnp.float32),
                    _pad_t(cates, CATE_P, jnp.int32),
                    _pad_t(combs, COMB_P, jnp.int32),
                    cate_table.reshape(-1), comb_table.reshape(-1), wbc)
    return out.reshape(B, 1)
